# baseline (speedup 1.0000x reference)
import functools

import jax
import jax.numpy as jnp
from jax import lax
from jax.experimental import pallas as pl
from jax.experimental.pallas import tpu as pltpu
from jax.experimental.pallas import tpu_sc as plsc

NUM_CODES = 8192
CODE_DIM = 32
COMMITMENT_COST = 0.25

CODE_TILE = 2048
NCT = NUM_CODES // CODE_TILE


def _argmin_body(lat_ref, cb_ref, idx_ref, best_val, best_idx):
    j = pl.program_id(1)

    @pl.when(j == 0)
    def _():
        best_val[...] = jnp.full_like(best_val, jnp.inf)
        best_idx[...] = jnp.zeros_like(best_idx)

    lat = lat_ref[0]
    flat = lat.T
    cb = cb_ref[...]
    fn = jnp.sum(lat * lat, axis=0)
    cn = jnp.sum(cb ** 2, axis=1)
    m = lax.dot_general(flat, cb, (((1,), (1,)), ((), ())),
                        preferred_element_type=jnp.float32)
    d = (fn[:, None] - 2.0 * m) + cn[None, :]
    val = jnp.min(d, axis=1)
    iot = lax.broadcasted_iota(jnp.int32, d.shape, 1)
    loc = jnp.min(jnp.where(d == val[:, None], iot, NUM_CODES), axis=1)

    better = val < best_val[...]
    best_idx[...] = jnp.where(better, loc + j * CODE_TILE, best_idx[...])
    best_val[...] = jnp.where(better, val, best_val[...]).astype(
        jnp.bfloat16).astype(jnp.float32)

    @pl.when(j == NCT - 1)
    def _():
        idx_ref[0, 0, :] = best_idx[...]


def _argmin_indices(latents, codebook):
    B, D, T = latents.shape
    return pl.pallas_call(
        _argmin_body,
        grid=(B, NCT),
        in_specs=[
            pl.BlockSpec((1, D, T), lambda i, j: (i, 0, 0)),
            pl.BlockSpec((CODE_TILE, D), lambda i, j: (j, 0)),
        ],
        out_specs=pl.BlockSpec((1, 1, T), lambda i, j: (i, 0, 0)),
        out_shape=jax.ShapeDtypeStruct((B, 1, T), jnp.int32),
        scratch_shapes=[
            pltpu.VMEM((T,), jnp.float32),
            pltpu.VMEM((T,), jnp.int32),
        ],
        compiler_params=pltpu.CompilerParams(
            dimension_semantics=("arbitrary", "arbitrary"),
        ),
    )(latents, codebook)


def _make_sc_gather(n_rows, d):
    info = plsc.get_sparse_core_info()
    nw = info.num_cores * info.num_subcores
    chunk = 128
    chunks_total = n_rows // chunk
    chunks_per_w = chunks_total // nw
    mesh = plsc.VectorSubcoreMesh(core_axis_name="c", subcore_axis_name="s")

    @functools.partial(
        pl.kernel, mesh=mesh,
        out_type=jax.ShapeDtypeStruct((n_rows, d), jnp.float32),
        scratch_types=[
            pltpu.VMEM((chunk,), jnp.int32),
            pltpu.VMEM((chunk, d), jnp.float32),
            pltpu.SemaphoreType.DMA,
        ],
        compiler_params=pltpu.CompilerParams(use_tc_tiling_on_sc=False),
    )
    def gather(table_hbm, idx_hbm, out_hbm, idx_v, rows_v, sem):
        wid = lax.axis_index("s") * info.num_cores + lax.axis_index("c")
        for k in range(chunks_per_w):
            c = wid * chunks_per_w + k
            pltpu.sync_copy(idx_hbm.at[c], idx_v)
            pltpu.async_copy(table_hbm.at[idx_v], rows_v, sem).wait()
            pltpu.sync_copy(rows_v, out_hbm.at[pl.ds(c * chunk, chunk)])

    return gather


def _finish_body(lat_ref, q_ref, qst_ref, loss_ref, acc_ref):
    i = pl.program_id(0)

    @pl.when(i == 0)
    def _():
        acc_ref[0] = 0.0

    lat = lat_ref[0]
    q = q_ref[0].T
    diff = q - lat
    qst_ref[0] = lat + diff
    acc_ref[0] += jnp.sum(diff ** 2)

    @pl.when(i == pl.num_programs(0) - 1)
    def _():
        m = acc_ref[0] / (pl.num_programs(0) * lat.size)
        loss_ref[0] = m + COMMITMENT_COST * m


def _finish(latents, qflat):
    B, D, T = latents.shape
    return pl.pallas_call(
        _finish_body,
        grid=(B,),
        in_specs=[
            pl.BlockSpec((1, D, T), lambda i: (i, 0, 0)),
            pl.BlockSpec((1, T, D), lambda i: (i, 0, 0)),
        ],
        out_specs=[
            pl.BlockSpec((1, D, T), lambda i: (i, 0, 0)),
            pl.BlockSpec(memory_space=pltpu.SMEM),
        ],
        out_shape=[
            jax.ShapeDtypeStruct((B, D, T), jnp.float32),
            jax.ShapeDtypeStruct((1,), jnp.float32),
        ],
        scratch_shapes=[pltpu.SMEM((1,), jnp.float32)],
        compiler_params=pltpu.CompilerParams(
            dimension_semantics=("arbitrary",),
        ),
    )(latents, qflat.reshape(B, T, D))


def kernel(latents, codebook):
    B, D, T = latents.shape
    indices = _argmin_indices(latents, codebook).reshape(B, T)
    idx2d = indices.reshape(-1, 128)
    qflat = _make_sc_gather(B * T, D)(codebook, idx2d)
    quantized_st, loss = _finish(latents, qflat)
    return quantized_st, jnp.reshape(loss, ()), indices


# speedup vs baseline: 1.0340x; 1.0340x over previous
import functools

import jax
import jax.numpy as jnp
from jax import lax
from jax.experimental import pallas as pl
from jax.experimental.pallas import tpu as pltpu
from jax.experimental.pallas import tpu_sc as plsc

NUM_CODES = 8192
CODE_DIM = 32
COMMITMENT_COST = 0.25

CODE_TILE = 2048
NCT = NUM_CODES // CODE_TILE


def _argmin_body(lat_ref, cb_ref, idx_ref, best_val, best_idx):
    j = pl.program_id(1)

    @pl.when(j == 0)
    def _():
        best_val[...] = jnp.full_like(best_val, jnp.inf)
        best_idx[...] = jnp.zeros_like(best_idx)

    lat = lat_ref[0]
    flat2 = (lat + lat).T
    cb = cb_ref[...]
    fn = jnp.sum(lat * lat, axis=0)
    cn = jnp.sum(cb ** 2, axis=1)
    m2 = lax.dot_general(flat2, cb, (((1,), (1,)), ((), ())),
                         preferred_element_type=jnp.float32)
    d = (fn[:, None] - m2) + cn[None, :]
    val = jnp.min(d, axis=1)
    iot = lax.broadcasted_iota(jnp.int32, d.shape, 1)
    loc = jnp.min(jnp.where(d == val[:, None], iot, NUM_CODES), axis=1)

    better = val < best_val[...]
    best_idx[...] = jnp.where(better, loc + j * CODE_TILE, best_idx[...])
    best_val[...] = jnp.where(better, val, best_val[...]).astype(
        jnp.bfloat16).astype(jnp.float32)

    @pl.when(j == NCT - 1)
    def _():
        idx_ref[0, 0, :] = best_idx[...]


def _argmin_indices(latents, codebook):
    B, D, T = latents.shape
    return pl.pallas_call(
        _argmin_body,
        grid=(B, NCT),
        in_specs=[
            pl.BlockSpec((1, D, T), lambda i, j: (i, 0, 0)),
            pl.BlockSpec((CODE_TILE, D), lambda i, j: (j, 0)),
        ],
        out_specs=pl.BlockSpec((1, 1, T), lambda i, j: (i, 0, 0)),
        out_shape=jax.ShapeDtypeStruct((B, 1, T), jnp.int32),
        scratch_shapes=[
            pltpu.VMEM((T,), jnp.float32),
            pltpu.VMEM((T,), jnp.int32),
        ],
        compiler_params=pltpu.CompilerParams(
            dimension_semantics=("arbitrary", "arbitrary"),
        ),
    )(latents, codebook)


def _make_sc_gather(n_rows, d):
    info = plsc.get_sparse_core_info()
    nw = info.num_cores * info.num_subcores
    chunk = 128
    chunks_total = n_rows // chunk
    chunks_per_w = chunks_total // nw
    mesh = plsc.VectorSubcoreMesh(core_axis_name="c", subcore_axis_name="s")

    @functools.partial(
        pl.kernel, mesh=mesh,
        out_type=jax.ShapeDtypeStruct((n_rows, d), jnp.float32),
        scratch_types=[
            pltpu.VMEM((chunk,), jnp.int32),
            pltpu.VMEM((chunk, d), jnp.float32),
            pltpu.SemaphoreType.DMA,
        ],
        compiler_params=pltpu.CompilerParams(use_tc_tiling_on_sc=False),
    )
    def gather(table_hbm, idx_hbm, out_hbm, idx_v, rows_v, sem):
        wid = lax.axis_index("s") * info.num_cores + lax.axis_index("c")
        for k in range(chunks_per_w):
            c = wid * chunks_per_w + k
            pltpu.sync_copy(idx_hbm.at[c], idx_v)
            pltpu.async_copy(table_hbm.at[idx_v], rows_v, sem).wait()
            pltpu.sync_copy(rows_v, out_hbm.at[pl.ds(c * chunk, chunk)])

    return gather


def _finish_body(lat_ref, q_ref, qst_ref, loss_ref, acc_ref):
    i = pl.program_id(0)

    @pl.when(i == 0)
    def _():
        acc_ref[0] = 0.0

    lat = lat_ref[0]
    q = q_ref[0].T
    diff = q - lat
    qst_ref[0] = lat + diff
    acc_ref[0] += jnp.sum(diff ** 2)

    @pl.when(i == pl.num_programs(0) - 1)
    def _():
        m = acc_ref[0] / (pl.num_programs(0) * lat.size)
        loss_ref[0] = m + COMMITMENT_COST * m


def _finish(latents, qflat):
    B, D, T = latents.shape
    return pl.pallas_call(
        _finish_body,
        grid=(B,),
        in_specs=[
            pl.BlockSpec((1, D, T), lambda i: (i, 0, 0)),
            pl.BlockSpec((1, T, D), lambda i: (i, 0, 0)),
        ],
        out_specs=[
            pl.BlockSpec((1, D, T), lambda i: (i, 0, 0)),
            pl.BlockSpec(memory_space=pltpu.SMEM),
        ],
        out_shape=[
            jax.ShapeDtypeStruct((B, D, T), jnp.float32),
            jax.ShapeDtypeStruct((1,), jnp.float32),
        ],
        scratch_shapes=[pltpu.SMEM((1,), jnp.float32)],
        compiler_params=pltpu.CompilerParams(
            dimension_semantics=("arbitrary",),
        ),
    )(latents, qflat.reshape(B, T, D))


def kernel(latents, codebook):
    B, D, T = latents.shape
    indices = _argmin_indices(latents, codebook).reshape(B, T)
    idx2d = indices.reshape(-1, 128)
    qflat = _make_sc_gather(B * T, D)(codebook, idx2d)
    quantized_st, loss = _finish(latents, qflat)
    return quantized_st, jnp.reshape(loss, ()), indices


# speedup vs baseline: 1.1835x; 1.1446x over previous
import functools

import jax
import jax.numpy as jnp
from jax import lax
from jax.experimental import pallas as pl
from jax.experimental.pallas import tpu as pltpu
from jax.experimental.pallas import tpu_sc as plsc

NUM_CODES = 8192
CODE_DIM = 32
COMMITMENT_COST = 0.25

CODE_TILE = 2048
NCT = NUM_CODES // CODE_TILE


def _argmin_body(lat_ref, cb_ref, idx_ref, best_val, best_idx):
    j = pl.program_id(1)

    @pl.when(j == 0)
    def _():
        best_val[...] = jnp.full_like(best_val, jnp.inf)
        best_idx[...] = jnp.zeros_like(best_idx)

    lat = lat_ref[0]
    flat2 = (lat + lat).T
    cb = cb_ref[...]
    fn = jnp.sum(lat * lat, axis=0)
    cn = jnp.sum(cb ** 2, axis=1)
    m2 = lax.dot_general(flat2, cb, (((1,), (1,)), ((), ())),
                         preferred_element_type=jnp.float32)
    m2t = m2.T
    d = (fn[None, :] - m2t) + cn[:, None]
    val = jnp.min(d, axis=0)
    iot = lax.broadcasted_iota(jnp.int32, d.shape, 0)
    loc = jnp.min(jnp.where(d == val[None, :], iot, NUM_CODES), axis=0)

    better = val < best_val[...]
    best_idx[...] = jnp.where(better, loc + j * CODE_TILE, best_idx[...])
    best_val[...] = jnp.where(better, val, best_val[...]).astype(
        jnp.bfloat16).astype(jnp.float32)

    @pl.when(j == NCT - 1)
    def _():
        idx_ref[0, 0, :] = best_idx[...]


def _argmin_indices(latents, codebook):
    B, D, T = latents.shape
    return pl.pallas_call(
        _argmin_body,
        grid=(B, NCT),
        in_specs=[
            pl.BlockSpec((1, D, T), lambda i, j: (i, 0, 0)),
            pl.BlockSpec((CODE_TILE, D), lambda i, j: (j, 0)),
        ],
        out_specs=pl.BlockSpec((1, 1, T), lambda i, j: (i, 0, 0)),
        out_shape=jax.ShapeDtypeStruct((B, 1, T), jnp.int32),
        scratch_shapes=[
            pltpu.VMEM((T,), jnp.float32),
            pltpu.VMEM((T,), jnp.int32),
        ],
        compiler_params=pltpu.CompilerParams(
            dimension_semantics=("arbitrary", "arbitrary"),
        ),
    )(latents, codebook)


def _make_sc_gather(n_rows, d):
    info = plsc.get_sparse_core_info()
    nw = info.num_cores * info.num_subcores
    chunk = 128
    chunks_total = n_rows // chunk
    chunks_per_w = chunks_total // nw
    mesh = plsc.VectorSubcoreMesh(core_axis_name="c", subcore_axis_name="s")

    @functools.partial(
        pl.kernel, mesh=mesh,
        out_type=jax.ShapeDtypeStruct((n_rows, d), jnp.float32),
        scratch_types=[
            pltpu.VMEM((chunk,), jnp.int32),
            pltpu.VMEM((chunk, d), jnp.float32),
            pltpu.SemaphoreType.DMA,
        ],
        compiler_params=pltpu.CompilerParams(use_tc_tiling_on_sc=False),
    )
    def gather(table_hbm, idx_hbm, out_hbm, idx_v, rows_v, sem):
        wid = lax.axis_index("s") * info.num_cores + lax.axis_index("c")
        for k in range(chunks_per_w):
            c = wid * chunks_per_w + k
            pltpu.sync_copy(idx_hbm.at[c], idx_v)
            pltpu.async_copy(table_hbm.at[idx_v], rows_v, sem).wait()
            pltpu.sync_copy(rows_v, out_hbm.at[pl.ds(c * chunk, chunk)])

    return gather


def _finish_body(lat_ref, q_ref, qst_ref, loss_ref, acc_ref):
    i = pl.program_id(0)

    @pl.when(i == 0)
    def _():
        acc_ref[0] = 0.0

    lat = lat_ref[0]
    q = q_ref[0].T
    diff = q - lat
    qst_ref[0] = lat + diff
    acc_ref[0] += jnp.sum(diff ** 2)

    @pl.when(i == pl.num_programs(0) - 1)
    def _():
        m = acc_ref[0] / (pl.num_programs(0) * lat.size)
        loss_ref[0] = m + COMMITMENT_COST * m


def _finish(latents, qflat):
    B, D, T = latents.shape
    return pl.pallas_call(
        _finish_body,
        grid=(B,),
        in_specs=[
            pl.BlockSpec((1, D, T), lambda i: (i, 0, 0)),
            pl.BlockSpec((1, T, D), lambda i: (i, 0, 0)),
        ],
        out_specs=[
            pl.BlockSpec((1, D, T), lambda i: (i, 0, 0)),
            pl.BlockSpec(memory_space=pltpu.SMEM),
        ],
        out_shape=[
            jax.ShapeDtypeStruct((B, D, T), jnp.float32),
            jax.ShapeDtypeStruct((1,), jnp.float32),
        ],
        scratch_shapes=[pltpu.SMEM((1,), jnp.float32)],
        compiler_params=pltpu.CompilerParams(
            dimension_semantics=("arbitrary",),
        ),
    )(latents, qflat.reshape(B, T, D))


def kernel(latents, codebook):
    B, D, T = latents.shape
    indices = _argmin_indices(latents, codebook).reshape(B, T)
    idx2d = indices.reshape(-1, 128)
    qflat = _make_sc_gather(B * T, D)(codebook, idx2d)
    quantized_st, loss = _finish(latents, qflat)
    return quantized_st, jnp.reshape(loss, ()), indices
